# baseline (speedup 1.0000x reference)
import functools

import jax
import jax.numpy as jnp
from jax import lax
from jax.experimental import pallas as pl
from jax.experimental.pallas import tpu as pltpu
from jax.experimental.pallas import tpu_sc as plsc

N = 10000
E = 320000
B = 64
NP = 10240
NT = 2 * NP
NSUB = 16
NCORE = 2
ET = E // NSUB
K = 64
EPT = 20480
NCHUNK = EPT // K
RPT = NP // NSUB


def _sc_mesh():
    return plsc.VectorSubcoreMesh(core_axis_name="c", subcore_axis_name="s")


def _zero_rows(buf, nrows, ncols):
    z = jnp.zeros((16,), jnp.float32)

    def body(i, _):
        for j in range(ncols // 16):
            buf[i, pl.ds(16 * j, 16)] = z
        return 0

    lax.fori_loop(0, nrows, body, 0)


IDXG = 16
NGRP = NCHUNK // IDXG
NBUF = 4


def _make_sc_scatter(F):
    nbuf = NBUF

    @functools.partial(
        pl.kernel,
        mesh=_sc_mesh(),
        out_type=jax.ShapeDtypeStruct((NT, F), jnp.float32),
        scratch_types=[
            [pltpu.VMEM((IDXG, K), jnp.int32) for _ in range(2)],
            [pltpu.VMEM((IDXG, K), jnp.int32) for _ in range(2)],
            [pltpu.VMEM((K, F), jnp.float32) for _ in range(nbuf)],
            [pltpu.SemaphoreType.DMA for _ in range(nbuf)],
            [pltpu.SemaphoreType.DMA for _ in range(nbuf)],
            [pltpu.SemaphoreType.DMA for _ in range(2)],
            pltpu.VMEM_SHARED((NP, F), jnp.float32),
        ],
        compiler_params=pltpu.CompilerParams(use_tc_tiling_on_sc=False),
        name=f"sc_edge_scatter_f{F}",
    )
    def k(g_hbm, src_hbm, dst_hbm, out_hbm, sidx, didx, rows, gsem, ssem,
          isem, acc):
        c = lax.axis_index("c")
        s = lax.axis_index("s")
        w = c * NSUB + s
        cbase = w * NCHUNK

        def idescs(gr, ib):
            off = cbase + gr * IDXG
            return (
                pltpu.make_async_copy(src_hbm.at[pl.ds(off, IDXG), :],
                                      sidx[ib], isem[ib]),
                pltpu.make_async_copy(dst_hbm.at[pl.ds(off, IDXG), :],
                                      didx[ib], isem[ib]),
            )

        for ib in range(2):
            for d_ in idescs(ib, ib):
                d_.start()

        _zero_rows(rows[0], K, F)
        rbase = s * RPT
        for j in range(RPT // K):
            pltpu.sync_copy(rows[0], acc.at[pl.ds(rbase + j * K, K), :])
        plsc.subcore_barrier()

        def gdesc(j, b, ib):
            return pltpu.make_async_copy(g_hbm.at[sidx[ib].at[j]], rows[b],
                                         gsem[b])

        def sdesc(j, b, ib):
            return pltpu.make_async_copy(rows[b], acc.at[didx[ib].at[j]],
                                         ssem[b])

        def rungroup(gr, ib):
            for d_ in idescs(gr, ib):
                d_.wait()
            nxt = gr + 2

            @pl.when(nxt < NGRP)
            def _():
                for d_ in idescs(nxt, ib):
                    d_.start()

            for j in range(nbuf):
                gdesc(j, j, ib).start()
            for j in range(IDXG):
                b = j % nbuf
                gdesc(j, b, ib).wait()
                pltpu.async_copy(rows[b], acc.at[didx[ib].at[j]], ssem[b],
                                 add=True)
                if j >= 1:
                    pb = (j - 1) % nbuf
                    sdesc(j - 1, pb, ib).wait()
                    if j - 1 + nbuf < IDXG:
                        gdesc(j - 1 + nbuf, pb, ib).start()
            sdesc(IDXG - 1, (IDXG - 1) % nbuf, ib).wait()

        def body(i, _):
            rungroup(2 * i, 0)
            rungroup(2 * i + 1, 1)
            return 0

        lax.fori_loop(0, NGRP // 2, body, 0)
        plsc.subcore_barrier()
        obase = c * NP + rbase
        for j in range(RPT // K):
            pltpu.sync_copy(acc.at[pl.ds(rbase + j * K, K), :], rows[j % 2])
            pltpu.sync_copy(rows[j % 2], out_hbm.at[pl.ds(obase + j * K, K), :])

    return k


DEGQ = 8


@functools.partial(
    pl.kernel,
    mesh=_sc_mesh(),
    out_type=jax.ShapeDtypeStruct((NT,), jnp.float32),
    scratch_types=[
        pltpu.VMEM((NCHUNK, K), jnp.int32),
        pltpu.VMEM((K,), jnp.float32),
        pltpu.VMEM((RPT,), jnp.float32),
        pltpu.SemaphoreType.DMA,
        pltpu.VMEM_SHARED((NP,), jnp.float32),
    ],
    compiler_params=pltpu.CompilerParams(use_tc_tiling_on_sc=False),
    name="sc_degree",
)
def _sc_degree(dst_hbm, out_hbm, dst_all, ones_v, wb, sem, acc):
    c = lax.axis_index("c")
    s = lax.axis_index("s")
    w = c * NSUB + s
    one = jnp.ones((16,), jnp.float32)
    zero = jnp.zeros((16,), jnp.float32)
    pltpu.sync_copy(dst_hbm.at[pl.ds(w * NCHUNK, NCHUNK), :], dst_all)
    for j in range(K // 16):
        ones_v[pl.ds(16 * j, 16)] = one

    def zbody(i, _):
        wb[pl.ds(16 * i, 16)] = zero
        return 0

    lax.fori_loop(0, RPT // 16, zbody, 0)
    rbase = s * RPT
    pltpu.sync_copy(wb, acc.at[pl.ds(rbase, RPT)])
    plsc.subcore_barrier()

    def sdesc(t):
        return pltpu.make_async_copy(ones_v, acc.at[dst_all.at[t]], sem)

    def body(gi, _):
        for b in range(DEGQ):
            pltpu.async_copy(ones_v, acc.at[dst_all.at[gi * DEGQ + b]], sem,
                             add=True)
        for b in range(DEGQ):
            sdesc(gi * DEGQ + b).wait()
        return 0

    lax.fori_loop(0, NCHUNK // DEGQ, body, 0)
    plsc.subcore_barrier()
    pltpu.sync_copy(acc.at[pl.ds(rbase, RPT)], wb)
    pltpu.sync_copy(wb, out_hbm.at[pl.ds(c * NP + rbase, RPT)])


BR = 2048


def _tc_first(x, W, deg):
    Fi, Fo = W.shape

    def body(x_ref, w_ref, deg_ref, g_ref, dis_ref):
        dis = lax.rsqrt(jnp.maximum(deg_ref[...] + 1.0, 1.0))
        g_ref[...] = dis * jnp.dot(x_ref[...], w_ref[...],
                                   preferred_element_type=jnp.float32)
        dis_ref[...] = dis

    return pl.pallas_call(
        body,
        grid=(NT // BR,),
        in_specs=[
            pl.BlockSpec((BR, Fi), lambda i: (i, 0)),
            pl.BlockSpec((Fi, Fo), lambda i: (0, 0)),
            pl.BlockSpec((BR, 1), lambda i: (i, 0)),
        ],
        out_specs=[
            pl.BlockSpec((BR, Fo), lambda i: (i, 0)),
            pl.BlockSpec((BR, 1), lambda i: (i, 0)),
        ],
        out_shape=[
            jax.ShapeDtypeStruct((NT, Fo), jnp.float32),
            jax.ShapeDtypeStruct((NT, 1), jnp.float32),
        ],
    )(x, W, deg)


def _tc_layer(acc, g, dis, b, W):
    Fi, Fo = W.shape

    def body(acc_ref, g_ref, dis_ref, b_ref, w_ref, out_ref):
        dis = dis_ref[...]
        h = jnp.maximum(dis * (acc_ref[...] + g_ref[...]) + b_ref[...], 0.0)
        out_ref[...] = dis * jnp.dot(h, w_ref[...],
                                     preferred_element_type=jnp.float32)

    return pl.pallas_call(
        body,
        grid=(NT // BR,),
        in_specs=[
            pl.BlockSpec((BR, Fi), lambda i: (i, 0)),
            pl.BlockSpec((BR, Fi), lambda i: (i, 0)),
            pl.BlockSpec((BR, 1), lambda i: (i, 0)),
            pl.BlockSpec((1, Fi), lambda i: (0, 0)),
            pl.BlockSpec((Fi, Fo), lambda i: (0, 0)),
        ],
        out_specs=pl.BlockSpec((BR, Fo), lambda i: (i, 0)),
        out_shape=jax.ShapeDtypeStruct((NT, Fo), jnp.float32),
    )(acc, g, dis, b, W)


def _tc_head(acc, g, dis, b3, batchp, Wa, Wt2, VtT, bt, Wf, bf, Ws, bs, R, T):
    def body(acc_ref, g_ref, dis_ref, b3_ref, batch_ref, wa_ref, wt2_ref,
             vtt_ref, bt_ref, wf_ref, bf_ref, ws_ref, bs_ref, r_ref, t_ref,
             out_ref):
        a = dis_ref[...] * (acc_ref[...] + g_ref[...]) + b3_ref[...]
        seg = lax.broadcasted_iota(jnp.int32, (2 * B, NT), 0)
        oh = (seg == batch_ref[...]).astype(jnp.float32)
        sums = jnp.dot(oh, a, preferred_element_type=jnp.float32)
        cnt = jnp.sum(oh, axis=1, keepdims=True)
        mean = sums / jnp.maximum(cnt, 1.0)
        ctx = jnp.tanh(jnp.dot(mean, wa_ref[...],
                               preferred_element_type=jnp.float32))
        ctx_pn = lax.dot_general(oh, ctx, (((0,), (0,)), ((), ())),
                                 preferred_element_type=jnp.float32)
        score = jax.nn.sigmoid(jnp.sum(a * ctx_pn, axis=1, keepdims=True))
        pooled = jnp.dot(oh, a * score, preferred_element_type=jnp.float32)
        e1 = pooled[:B]
        e2 = pooled[B:]
        op = (jnp.dot(e1, r_ref[...], preferred_element_type=jnp.float32)
              * jnp.dot(e2, t_ref[...], preferred_element_type=jnp.float32))
        scoring = jnp.dot(op, wt2_ref[...], preferred_element_type=jnp.float32)
        blk = jnp.dot(jnp.concatenate([e1, e2], axis=1), vtt_ref[...],
                      preferred_element_type=jnp.float32)
        s = jnp.maximum(scoring + blk + bt_ref[...], 0.0)
        s = jnp.maximum(jnp.dot(s, wf_ref[...],
                                preferred_element_type=jnp.float32)
                        + bf_ref[...], 0.0)
        out_ref[...] = jax.nn.sigmoid(
            jnp.dot(s, ws_ref[...], preferred_element_type=jnp.float32)
            + bs_ref[...])

    return pl.pallas_call(
        body,
        out_shape=jax.ShapeDtypeStruct((B, 1), jnp.float32),
    )(acc, g, dis, b3, batchp, Wa, Wt2, VtT, bt, Wf, bf, Ws, bs, R, T)


def kernel(features_1, edge_index_1, features_2, edge_index_2, batch_1,
           batch_2, W1, b1, W2, b2, W3, b3, Wa, Wt, Vt, bt, Wf, bf, Ws, bs):
    x = jnp.concatenate([
        jnp.pad(features_1, ((0, NP - N), (0, 0))),
        jnp.pad(features_2, ((0, NP - N), (0, 0))),
    ], axis=0)

    padn = EPT - ET
    pad_dst = (jnp.arange(padn, dtype=jnp.int32) % (NP - N)) + N

    def make_edges(ei, gidx):
        srcs = ei[0].reshape(NSUB, ET) + gidx * NP
        dsts = ei[1].reshape(NSUB, ET)
        srcs = jnp.concatenate(
            [srcs, jnp.broadcast_to(pad_dst + gidx * NP, (NSUB, padn))], axis=1)
        dsts = jnp.concatenate(
            [dsts, jnp.broadcast_to(pad_dst, (NSUB, padn))], axis=1)
        return srcs, dsts

    s1, d1 = make_edges(edge_index_1, 0)
    s2, d2 = make_edges(edge_index_2, 1)
    src_idx = jnp.stack([s1, s2]).reshape(2 * NSUB * NCHUNK, K)
    dst_idx = jnp.stack([d1, d2]).reshape(2 * NSUB * NCHUNK, K)

    batchp = jnp.concatenate([
        jnp.pad(batch_1, (0, NP - N), constant_values=-1),
        jnp.pad(batch_2 + B, (0, NP - N), constant_values=-1),
    ]).reshape(1, NT)

    Wt2 = Wt.reshape(32 * 32, 16)
    VtT = Vt.T
    ar = jnp.arange(1024, dtype=jnp.int32)
    a32 = jnp.arange(32, dtype=jnp.int32)
    R = (ar[None, :] // 32 == a32[:, None]).astype(jnp.float32)
    T = (ar[None, :] % 32 == a32[:, None]).astype(jnp.float32)

    deg = _sc_degree(dst_idx).reshape(NT, 1)
    g1, dis = _tc_first(x, W1, deg)
    acc1 = _make_sc_scatter(128)(g1, src_idx, dst_idx)
    g2 = _tc_layer(acc1, g1, dis, b1.reshape(1, -1), W2)
    acc2 = _make_sc_scatter(64)(g2, src_idx, dst_idx)
    g3 = _tc_layer(acc2, g2, dis, b2.reshape(1, -1), W3)
    acc3 = _make_sc_scatter(32)(g3, src_idx, dst_idx)
    score = _tc_head(acc3, g3, dis, b3.reshape(1, -1), batchp, Wa, Wt2, VtT,
                     bt.reshape(1, -1), Wf, bf.reshape(1, -1), Ws,
                     bs.reshape(1, -1), R, T)
    return score.reshape(-1)


# speedup vs baseline: 1.0169x; 1.0169x over previous
import functools

import jax
import jax.numpy as jnp
from jax import lax
from jax.experimental import pallas as pl
from jax.experimental.pallas import tpu as pltpu
from jax.experimental.pallas import tpu_sc as plsc

N = 10000
E = 320000
B = 64
NP = 10240
NT = 2 * NP
NSUB = 16
NCORE = 2
ET = E // NSUB
K = 128
EPT = 20480
NCHUNK = EPT // K
RPT = NP // NSUB


def _sc_mesh():
    return plsc.VectorSubcoreMesh(core_axis_name="c", subcore_axis_name="s")


def _zero_rows(buf, nrows, ncols):
    z = jnp.zeros((16,), jnp.float32)

    def body(i, _):
        for j in range(ncols // 16):
            buf[i, pl.ds(16 * j, 16)] = z
        return 0

    lax.fori_loop(0, nrows, body, 0)


IDXG = 16
NGRP = NCHUNK // IDXG


def _make_sc_scatter(F):
    nbuf = 2 if F == 128 else 4

    @functools.partial(
        pl.kernel,
        mesh=_sc_mesh(),
        out_type=jax.ShapeDtypeStruct((NT, F), jnp.float32),
        scratch_types=[
            [pltpu.VMEM((IDXG, K), jnp.int32) for _ in range(2)],
            [pltpu.VMEM((IDXG, K), jnp.int32) for _ in range(2)],
            [pltpu.VMEM((K, F), jnp.float32) for _ in range(nbuf)],
            [pltpu.SemaphoreType.DMA for _ in range(nbuf)],
            [pltpu.SemaphoreType.DMA for _ in range(nbuf)],
            [pltpu.SemaphoreType.DMA for _ in range(2)],
            pltpu.VMEM_SHARED((NP, F), jnp.float32),
        ],
        compiler_params=pltpu.CompilerParams(use_tc_tiling_on_sc=False),
        name=f"sc_edge_scatter_f{F}",
    )
    def k(g_hbm, src_hbm, dst_hbm, out_hbm, sidx, didx, rows, gsem, ssem,
          isem, acc):
        c = lax.axis_index("c")
        s = lax.axis_index("s")
        w = c * NSUB + s
        cbase = w * NCHUNK

        def idescs(gr, ib):
            off = cbase + gr * IDXG
            return (
                pltpu.make_async_copy(src_hbm.at[pl.ds(off, IDXG), :],
                                      sidx[ib], isem[ib]),
                pltpu.make_async_copy(dst_hbm.at[pl.ds(off, IDXG), :],
                                      didx[ib], isem[ib]),
            )

        for ib in range(2):
            for d_ in idescs(ib, ib):
                d_.start()

        _zero_rows(rows[0], K, F)
        rbase = s * RPT
        for j in range(RPT // K):
            pltpu.sync_copy(rows[0], acc.at[pl.ds(rbase + j * K, K), :])
        plsc.subcore_barrier()

        def gdesc(j, b, ib):
            return pltpu.make_async_copy(g_hbm.at[sidx[ib].at[j]], rows[b],
                                         gsem[b])

        def sdesc(j, b, ib):
            return pltpu.make_async_copy(rows[b], acc.at[didx[ib].at[j]],
                                         ssem[b])

        def rungroup(gr, ib):
            for d_ in idescs(gr, ib):
                d_.wait()
            nxt = gr + 2

            @pl.when(nxt < NGRP)
            def _():
                for d_ in idescs(nxt, ib):
                    d_.start()

            for j in range(nbuf):
                gdesc(j, j, ib).start()
            for j in range(IDXG):
                b = j % nbuf
                gdesc(j, b, ib).wait()
                pltpu.async_copy(rows[b], acc.at[didx[ib].at[j]], ssem[b],
                                 add=True)
                if j >= 1:
                    pb = (j - 1) % nbuf
                    sdesc(j - 1, pb, ib).wait()
                    if j - 1 + nbuf < IDXG:
                        gdesc(j - 1 + nbuf, pb, ib).start()
            sdesc(IDXG - 1, (IDXG - 1) % nbuf, ib).wait()

        def body(i, _):
            rungroup(2 * i, 0)
            rungroup(2 * i + 1, 1)
            return 0

        lax.fori_loop(0, NGRP // 2, body, 0)
        plsc.subcore_barrier()
        obase = c * NP + rbase
        for j in range(RPT // K):
            pltpu.sync_copy(acc.at[pl.ds(rbase + j * K, K), :], rows[j % 2])
            pltpu.sync_copy(rows[j % 2], out_hbm.at[pl.ds(obase + j * K, K), :])

    return k


DEGQ = 8


@functools.partial(
    pl.kernel,
    mesh=_sc_mesh(),
    out_type=jax.ShapeDtypeStruct((NT,), jnp.float32),
    scratch_types=[
        pltpu.VMEM((NCHUNK, K), jnp.int32),
        pltpu.VMEM((K,), jnp.float32),
        pltpu.VMEM((RPT,), jnp.float32),
        pltpu.SemaphoreType.DMA,
        pltpu.VMEM_SHARED((NP,), jnp.float32),
    ],
    compiler_params=pltpu.CompilerParams(use_tc_tiling_on_sc=False),
    name="sc_degree",
)
def _sc_degree(dst_hbm, out_hbm, dst_all, ones_v, wb, sem, acc):
    c = lax.axis_index("c")
    s = lax.axis_index("s")
    w = c * NSUB + s
    one = jnp.ones((16,), jnp.float32)
    zero = jnp.zeros((16,), jnp.float32)
    pltpu.sync_copy(dst_hbm.at[pl.ds(w * NCHUNK, NCHUNK), :], dst_all)
    for j in range(K // 16):
        ones_v[pl.ds(16 * j, 16)] = one

    def zbody(i, _):
        wb[pl.ds(16 * i, 16)] = zero
        return 0

    lax.fori_loop(0, RPT // 16, zbody, 0)
    rbase = s * RPT
    pltpu.sync_copy(wb, acc.at[pl.ds(rbase, RPT)])
    plsc.subcore_barrier()

    def sdesc(t):
        return pltpu.make_async_copy(ones_v, acc.at[dst_all.at[t]], sem)

    def body(gi, _):
        for b in range(DEGQ):
            pltpu.async_copy(ones_v, acc.at[dst_all.at[gi * DEGQ + b]], sem,
                             add=True)
        for b in range(DEGQ):
            sdesc(gi * DEGQ + b).wait()
        return 0

    lax.fori_loop(0, NCHUNK // DEGQ, body, 0)
    plsc.subcore_barrier()
    pltpu.sync_copy(acc.at[pl.ds(rbase, RPT)], wb)
    pltpu.sync_copy(wb, out_hbm.at[pl.ds(c * NP + rbase, RPT)])


BR = 2048


def _tc_first(x, W, deg):
    Fi, Fo = W.shape

    def body(x_ref, w_ref, deg_ref, g_ref, dis_ref):
        dis = lax.rsqrt(jnp.maximum(deg_ref[...] + 1.0, 1.0))
        g_ref[...] = dis * jnp.dot(x_ref[...], w_ref[...],
                                   preferred_element_type=jnp.float32)
        dis_ref[...] = dis

    return pl.pallas_call(
        body,
        grid=(NT // BR,),
        in_specs=[
            pl.BlockSpec((BR, Fi), lambda i: (i, 0)),
            pl.BlockSpec((Fi, Fo), lambda i: (0, 0)),
            pl.BlockSpec((BR, 1), lambda i: (i, 0)),
        ],
        out_specs=[
            pl.BlockSpec((BR, Fo), lambda i: (i, 0)),
            pl.BlockSpec((BR, 1), lambda i: (i, 0)),
        ],
        out_shape=[
            jax.ShapeDtypeStruct((NT, Fo), jnp.float32),
            jax.ShapeDtypeStruct((NT, 1), jnp.float32),
        ],
    )(x, W, deg)


def _tc_layer(acc, g, dis, b, W):
    Fi, Fo = W.shape

    def body(acc_ref, g_ref, dis_ref, b_ref, w_ref, out_ref):
        dis = dis_ref[...]
        h = jnp.maximum(dis * (acc_ref[...] + g_ref[...]) + b_ref[...], 0.0)
        out_ref[...] = dis * jnp.dot(h, w_ref[...],
                                     preferred_element_type=jnp.float32)

    return pl.pallas_call(
        body,
        grid=(NT // BR,),
        in_specs=[
            pl.BlockSpec((BR, Fi), lambda i: (i, 0)),
            pl.BlockSpec((BR, Fi), lambda i: (i, 0)),
            pl.BlockSpec((BR, 1), lambda i: (i, 0)),
            pl.BlockSpec((1, Fi), lambda i: (0, 0)),
            pl.BlockSpec((Fi, Fo), lambda i: (0, 0)),
        ],
        out_specs=pl.BlockSpec((BR, Fo), lambda i: (i, 0)),
        out_shape=jax.ShapeDtypeStruct((NT, Fo), jnp.float32),
    )(acc, g, dis, b, W)


def _tc_head(acc, g, dis, b3, batchp, Wa, Wt2, VtT, bt, Wf, bf, Ws, bs, R, T):
    def body(acc_ref, g_ref, dis_ref, b3_ref, batch_ref, wa_ref, wt2_ref,
             vtt_ref, bt_ref, wf_ref, bf_ref, ws_ref, bs_ref, r_ref, t_ref,
             out_ref):
        a = dis_ref[...] * (acc_ref[...] + g_ref[...]) + b3_ref[...]
        seg = lax.broadcasted_iota(jnp.int32, (2 * B, NT), 0)
        oh = (seg == batch_ref[...]).astype(jnp.float32)
        sums = jnp.dot(oh, a, preferred_element_type=jnp.float32)
        cnt = jnp.sum(oh, axis=1, keepdims=True)
        mean = sums / jnp.maximum(cnt, 1.0)
        ctx = jnp.tanh(jnp.dot(mean, wa_ref[...],
                               preferred_element_type=jnp.float32))
        ctx_pn = lax.dot_general(oh, ctx, (((0,), (0,)), ((), ())),
                                 preferred_element_type=jnp.float32)
        score = jax.nn.sigmoid(jnp.sum(a * ctx_pn, axis=1, keepdims=True))
        pooled = jnp.dot(oh, a * score, preferred_element_type=jnp.float32)
        e1 = pooled[:B]
        e2 = pooled[B:]
        op = (jnp.dot(e1, r_ref[...], preferred_element_type=jnp.float32)
              * jnp.dot(e2, t_ref[...], preferred_element_type=jnp.float32))
        scoring = jnp.dot(op, wt2_ref[...], preferred_element_type=jnp.float32)
        blk = jnp.dot(jnp.concatenate([e1, e2], axis=1), vtt_ref[...],
                      preferred_element_type=jnp.float32)
        s = jnp.maximum(scoring + blk + bt_ref[...], 0.0)
        s = jnp.maximum(jnp.dot(s, wf_ref[...],
                                preferred_element_type=jnp.float32)
                        + bf_ref[...], 0.0)
        out_ref[...] = jax.nn.sigmoid(
            jnp.dot(s, ws_ref[...], preferred_element_type=jnp.float32)
            + bs_ref[...])

    return pl.pallas_call(
        body,
        out_shape=jax.ShapeDtypeStruct((B, 1), jnp.float32),
    )(acc, g, dis, b3, batchp, Wa, Wt2, VtT, bt, Wf, bf, Ws, bs, R, T)


def kernel(features_1, edge_index_1, features_2, edge_index_2, batch_1,
           batch_2, W1, b1, W2, b2, W3, b3, Wa, Wt, Vt, bt, Wf, bf, Ws, bs):
    x = jnp.concatenate([
        jnp.pad(features_1, ((0, NP - N), (0, 0))),
        jnp.pad(features_2, ((0, NP - N), (0, 0))),
    ], axis=0)

    padn = EPT - ET
    pad_dst = (jnp.arange(padn, dtype=jnp.int32) % (NP - N)) + N

    def make_edges(ei, gidx):
        srcs = ei[0].reshape(NSUB, ET) + gidx * NP
        dsts = ei[1].reshape(NSUB, ET)
        srcs = jnp.concatenate(
            [srcs, jnp.broadcast_to(pad_dst + gidx * NP, (NSUB, padn))], axis=1)
        dsts = jnp.concatenate(
            [dsts, jnp.broadcast_to(pad_dst, (NSUB, padn))], axis=1)
        return srcs, dsts

    s1, d1 = make_edges(edge_index_1, 0)
    s2, d2 = make_edges(edge_index_2, 1)
    src_idx = jnp.stack([s1, s2]).reshape(2 * NSUB * NCHUNK, K)
    dst_idx = jnp.stack([d1, d2]).reshape(2 * NSUB * NCHUNK, K)

    batchp = jnp.concatenate([
        jnp.pad(batch_1, (0, NP - N), constant_values=-1),
        jnp.pad(batch_2 + B, (0, NP - N), constant_values=-1),
    ]).reshape(1, NT)

    Wt2 = Wt.reshape(32 * 32, 16)
    VtT = Vt.T
    ar = jnp.arange(1024, dtype=jnp.int32)
    a32 = jnp.arange(32, dtype=jnp.int32)
    R = (ar[None, :] // 32 == a32[:, None]).astype(jnp.float32)
    T = (ar[None, :] % 32 == a32[:, None]).astype(jnp.float32)

    deg = _sc_degree(dst_idx).reshape(NT, 1)
    g1, dis = _tc_first(x, W1, deg)
    acc1 = _make_sc_scatter(128)(g1, src_idx, dst_idx)
    g2 = _tc_layer(acc1, g1, dis, b1.reshape(1, -1), W2)
    acc2 = _make_sc_scatter(64)(g2, src_idx, dst_idx)
    g3 = _tc_layer(acc2, g2, dis, b2.reshape(1, -1), W3)
    acc3 = _make_sc_scatter(32)(g3, src_idx, dst_idx)
    score = _tc_head(acc3, g3, dis, b3.reshape(1, -1), batchp, Wa, Wt2, VtT,
                     bt.reshape(1, -1), Wf, bf.reshape(1, -1), Ws,
                     bs.reshape(1, -1), R, T)
    return score.reshape(-1)


# speedup vs baseline: 1.1237x; 1.1051x over previous
import functools

import jax
import jax.numpy as jnp
from jax import lax
from jax.experimental import pallas as pl
from jax.experimental.pallas import tpu as pltpu
from jax.experimental.pallas import tpu_sc as plsc

N = 10000
E = 320000
B = 64
NP = 10240
NT = 2 * NP
NSUB = 16
NCORE = 2
ET = E // NSUB
K = 128
EPT = 20480
NCHUNK = EPT // K
RPT = NP // NSUB


def _sc_mesh():
    return plsc.VectorSubcoreMesh(core_axis_name="c", subcore_axis_name="s")


def _zero_rows(buf, nrows, ncols):
    z = jnp.zeros((16,), jnp.float32)

    def body(i, _):
        for j in range(ncols // 16):
            buf[i, pl.ds(16 * j, 16)] = z
        return 0

    lax.fori_loop(0, nrows, body, 0)


IDXG = 16
NGRP = NCHUNK // IDXG


def _make_sc_scatter(F):
    nbuf = 2 if F == 128 else 4

    @functools.partial(
        pl.kernel,
        mesh=_sc_mesh(),
        out_type=jax.ShapeDtypeStruct((NT, F), jnp.float32),
        scratch_types=[
            [pltpu.VMEM((IDXG, K), jnp.int32) for _ in range(2)],
            [pltpu.VMEM((IDXG, K), jnp.int32) for _ in range(2)],
            [pltpu.VMEM((K, F), jnp.float32) for _ in range(nbuf)],
            [pltpu.SemaphoreType.DMA for _ in range(nbuf)],
            [pltpu.SemaphoreType.DMA for _ in range(2)],
            pltpu.VMEM_SHARED((NP, F), jnp.float32),
        ],
        compiler_params=pltpu.CompilerParams(use_tc_tiling_on_sc=False),
        name=f"sc_edge_scatter_f{F}",
    )
    def k(g_hbm, src_hbm, dst_hbm, out_hbm, sidx, didx, rows, gsem, isem, acc):
        c = lax.axis_index("c")
        s = lax.axis_index("s")
        w = c * NSUB + s
        cbase = w * NCHUNK

        def idescs(gr, ib):
            off = cbase + gr * IDXG
            return (
                pltpu.make_async_copy(src_hbm.at[pl.ds(off, IDXG), :],
                                      sidx[ib], isem[ib]),
                pltpu.make_async_copy(dst_hbm.at[pl.ds(off, IDXG), :],
                                      didx[ib], isem[ib]),
            )

        for ib in range(2):
            for d_ in idescs(ib, ib):
                d_.start()

        _zero_rows(rows[0], K, F)
        rbase = s * RPT
        for j in range(RPT // K):
            pltpu.sync_copy(rows[0], acc.at[pl.ds(rbase + j * K, K), :])
        plsc.subcore_barrier()

        def gdesc(j, b, ib):
            return pltpu.make_async_copy(g_hbm.at[sidx[ib].at[j]], rows[b],
                                         gsem[b])

        def rungroup(gr, ib):
            for d_ in idescs(gr, ib):
                d_.wait()
            nxt = gr + 2

            @pl.when(nxt < NGRP)
            def _():
                for d_ in idescs(nxt, ib):
                    d_.start()

            for j in range(nbuf):
                gdesc(j, j, ib).start()
            for j in range(IDXG):
                b = j % nbuf
                gdesc(j, b, ib).wait()
                pltpu.sync_copy(rows[b], acc.at[didx[ib].at[j]], add=True)
                if j + nbuf < IDXG:
                    gdesc(j + nbuf, b, ib).start()

        def body(i, _):
            rungroup(2 * i, 0)
            rungroup(2 * i + 1, 1)
            return 0

        lax.fori_loop(0, NGRP // 2, body, 0)
        plsc.subcore_barrier()
        obase = c * NP + rbase
        for j in range(RPT // K):
            pltpu.sync_copy(acc.at[pl.ds(rbase + j * K, K), :], rows[j % 2])
            pltpu.sync_copy(rows[j % 2], out_hbm.at[pl.ds(obase + j * K, K), :])

    return k


DEGQ = 8


@functools.partial(
    pl.kernel,
    mesh=_sc_mesh(),
    out_type=jax.ShapeDtypeStruct((NT,), jnp.float32),
    scratch_types=[
        pltpu.VMEM((NCHUNK, K), jnp.int32),
        pltpu.VMEM((K,), jnp.float32),
        pltpu.VMEM((RPT,), jnp.float32),
        pltpu.SemaphoreType.DMA,
        pltpu.VMEM_SHARED((NP,), jnp.float32),
    ],
    compiler_params=pltpu.CompilerParams(use_tc_tiling_on_sc=False),
    name="sc_degree",
)
def _sc_degree(dst_hbm, out_hbm, dst_all, ones_v, wb, sem, acc):
    c = lax.axis_index("c")
    s = lax.axis_index("s")
    w = c * NSUB + s
    one = jnp.ones((16,), jnp.float32)
    zero = jnp.zeros((16,), jnp.float32)
    pltpu.sync_copy(dst_hbm.at[pl.ds(w * NCHUNK, NCHUNK), :], dst_all)
    for j in range(K // 16):
        ones_v[pl.ds(16 * j, 16)] = one

    def zbody(i, _):
        wb[pl.ds(16 * i, 16)] = zero
        return 0

    lax.fori_loop(0, RPT // 16, zbody, 0)
    rbase = s * RPT
    pltpu.sync_copy(wb, acc.at[pl.ds(rbase, RPT)])
    plsc.subcore_barrier()

    def sdesc(t):
        return pltpu.make_async_copy(ones_v, acc.at[dst_all.at[t]], sem)

    def body(gi, _):
        for b in range(DEGQ):
            pltpu.async_copy(ones_v, acc.at[dst_all.at[gi * DEGQ + b]], sem,
                             add=True)
        for b in range(DEGQ):
            sdesc(gi * DEGQ + b).wait()
        return 0

    lax.fori_loop(0, NCHUNK // DEGQ, body, 0)
    plsc.subcore_barrier()
    pltpu.sync_copy(acc.at[pl.ds(rbase, RPT)], wb)
    pltpu.sync_copy(wb, out_hbm.at[pl.ds(c * NP + rbase, RPT)])


BR = 2048


def _tc_first(x, W, deg):
    Fi, Fo = W.shape

    def body(x_ref, w_ref, deg_ref, g_ref, dis_ref):
        dis = lax.rsqrt(jnp.maximum(deg_ref[...] + 1.0, 1.0))
        g_ref[...] = dis * jnp.dot(x_ref[...], w_ref[...],
                                   preferred_element_type=jnp.float32)
        dis_ref[...] = dis

    return pl.pallas_call(
        body,
        grid=(NT // BR,),
        in_specs=[
            pl.BlockSpec((BR, Fi), lambda i: (i, 0)),
            pl.BlockSpec((Fi, Fo), lambda i: (0, 0)),
            pl.BlockSpec((BR, 1), lambda i: (i, 0)),
        ],
        out_specs=[
            pl.BlockSpec((BR, Fo), lambda i: (i, 0)),
            pl.BlockSpec((BR, 1), lambda i: (i, 0)),
        ],
        out_shape=[
            jax.ShapeDtypeStruct((NT, Fo), jnp.float32),
            jax.ShapeDtypeStruct((NT, 1), jnp.float32),
        ],
    )(x, W, deg)


def _tc_layer(acc, g, dis, b, W):
    Fi, Fo = W.shape

    def body(acc_ref, g_ref, dis_ref, b_ref, w_ref, out_ref):
        dis = dis_ref[...]
        h = jnp.maximum(dis * (acc_ref[...] + g_ref[...]) + b_ref[...], 0.0)
        out_ref[...] = dis * jnp.dot(h, w_ref[...],
                                     preferred_element_type=jnp.float32)

    return pl.pallas_call(
        body,
        grid=(NT // BR,),
        in_specs=[
            pl.BlockSpec((BR, Fi), lambda i: (i, 0)),
            pl.BlockSpec((BR, Fi), lambda i: (i, 0)),
            pl.BlockSpec((BR, 1), lambda i: (i, 0)),
            pl.BlockSpec((1, Fi), lambda i: (0, 0)),
            pl.BlockSpec((Fi, Fo), lambda i: (0, 0)),
        ],
        out_specs=pl.BlockSpec((BR, Fo), lambda i: (i, 0)),
        out_shape=jax.ShapeDtypeStruct((NT, Fo), jnp.float32),
    )(acc, g, dis, b, W)


def _tc_head(acc, g, dis, b3, batchp, Wa, Wt2, VtT, bt, Wf, bf, Ws, bs, R, T):
    def body(acc_ref, g_ref, dis_ref, b3_ref, batch_ref, wa_ref, wt2_ref,
             vtt_ref, bt_ref, wf_ref, bf_ref, ws_ref, bs_ref, r_ref, t_ref,
             out_ref):
        a = dis_ref[...] * (acc_ref[...] + g_ref[...]) + b3_ref[...]
        seg = lax.broadcasted_iota(jnp.int32, (2 * B, NT), 0)
        oh = (seg == batch_ref[...]).astype(jnp.float32)
        sums = jnp.dot(oh, a, preferred_element_type=jnp.float32)
        cnt = jnp.sum(oh, axis=1, keepdims=True)
        mean = sums / jnp.maximum(cnt, 1.0)
        ctx = jnp.tanh(jnp.dot(mean, wa_ref[...],
                               preferred_element_type=jnp.float32))
        ctx_pn = lax.dot_general(oh, ctx, (((0,), (0,)), ((), ())),
                                 preferred_element_type=jnp.float32)
        score = jax.nn.sigmoid(jnp.sum(a * ctx_pn, axis=1, keepdims=True))
        pooled = jnp.dot(oh, a * score, preferred_element_type=jnp.float32)
        e1 = pooled[:B]
        e2 = pooled[B:]
        op = (jnp.dot(e1, r_ref[...], preferred_element_type=jnp.float32)
              * jnp.dot(e2, t_ref[...], preferred_element_type=jnp.float32))
        scoring = jnp.dot(op, wt2_ref[...], preferred_element_type=jnp.float32)
        blk = jnp.dot(jnp.concatenate([e1, e2], axis=1), vtt_ref[...],
                      preferred_element_type=jnp.float32)
        s = jnp.maximum(scoring + blk + bt_ref[...], 0.0)
        s = jnp.maximum(jnp.dot(s, wf_ref[...],
                                preferred_element_type=jnp.float32)
                        + bf_ref[...], 0.0)
        out_ref[...] = jax.nn.sigmoid(
            jnp.dot(s, ws_ref[...], preferred_element_type=jnp.float32)
            + bs_ref[...])

    return pl.pallas_call(
        body,
        out_shape=jax.ShapeDtypeStruct((B, 1), jnp.float32),
    )(acc, g, dis, b3, batchp, Wa, Wt2, VtT, bt, Wf, bf, Ws, bs, R, T)


def kernel(features_1, edge_index_1, features_2, edge_index_2, batch_1,
           batch_2, W1, b1, W2, b2, W3, b3, Wa, Wt, Vt, bt, Wf, bf, Ws, bs):
    x = jnp.concatenate([
        jnp.pad(features_1, ((0, NP - N), (0, 0))),
        jnp.pad(features_2, ((0, NP - N), (0, 0))),
    ], axis=0)

    padn = EPT - ET
    pad_dst = (jnp.arange(padn, dtype=jnp.int32) % (NP - N)) + N

    def make_edges(ei, gidx):
        srcs = ei[0].reshape(NSUB, ET) + gidx * NP
        dsts = ei[1].reshape(NSUB, ET)
        srcs = jnp.concatenate(
            [srcs, jnp.broadcast_to(pad_dst + gidx * NP, (NSUB, padn))], axis=1)
        dsts = jnp.concatenate(
            [dsts, jnp.broadcast_to(pad_dst, (NSUB, padn))], axis=1)
        return srcs, dsts

    s1, d1 = make_edges(edge_index_1, 0)
    s2, d2 = make_edges(edge_index_2, 1)
    src_idx = jnp.stack([s1, s2]).reshape(2 * NSUB * NCHUNK, K)
    dst_idx = jnp.stack([d1, d2]).reshape(2 * NSUB * NCHUNK, K)

    batchp = jnp.concatenate([
        jnp.pad(batch_1, (0, NP - N), constant_values=-1),
        jnp.pad(batch_2 + B, (0, NP - N), constant_values=-1),
    ]).reshape(1, NT)

    Wt2 = Wt.reshape(32 * 32, 16)
    VtT = Vt.T
    ar = jnp.arange(1024, dtype=jnp.int32)
    a32 = jnp.arange(32, dtype=jnp.int32)
    R = (ar[None, :] // 32 == a32[:, None]).astype(jnp.float32)
    T = (ar[None, :] % 32 == a32[:, None]).astype(jnp.float32)

    deg = _sc_degree(dst_idx).reshape(NT, 1)
    g1, dis = _tc_first(x, W1, deg)
    acc1 = _make_sc_scatter(128)(g1, src_idx, dst_idx)
    g2 = _tc_layer(acc1, g1, dis, b1.reshape(1, -1), W2)
    acc2 = _make_sc_scatter(64)(g2, src_idx, dst_idx)
    g3 = _tc_layer(acc2, g2, dis, b2.reshape(1, -1), W3)
    acc3 = _make_sc_scatter(32)(g3, src_idx, dst_idx)
    score = _tc_head(acc3, g3, dis, b3.reshape(1, -1), batchp, Wa, Wt2, VtT,
                     bt.reshape(1, -1), Wf, bf.reshape(1, -1), Ws,
                     bs.reshape(1, -1), R, T)
    return score.reshape(-1)


# speedup vs baseline: 1.4119x; 1.2565x over previous
import functools

import jax
import jax.numpy as jnp
from jax import lax
from jax.experimental import pallas as pl
from jax.experimental.pallas import tpu as pltpu
from jax.experimental.pallas import tpu_sc as plsc

N = 10000
E = 320000
B = 64
NP = 10240
NT = 2 * NP
NSUB = 16
NCORE = 2
ET = E // NSUB
K = 128
EPT = 20480
NCHUNK = EPT // K
RPT = NP // NSUB


def _sc_mesh():
    return plsc.VectorSubcoreMesh(core_axis_name="c", subcore_axis_name="s")


def _zero_rows(buf, nrows, ncols):
    z = jnp.zeros((32,), jnp.bfloat16)

    def body(i, _):
        for j in range(ncols // 32):
            buf[i, pl.ds(32 * j, 32)] = z
        return 0

    lax.fori_loop(0, nrows, body, 0)


IDXG = 16
NGRP = NCHUNK // IDXG


def _make_sc_scatter(F):
    nbuf = 4

    @functools.partial(
        pl.kernel,
        mesh=_sc_mesh(),
        out_type=jax.ShapeDtypeStruct((NT, F), jnp.bfloat16),
        scratch_types=[
            [pltpu.VMEM((IDXG, K), jnp.int32) for _ in range(2)],
            [pltpu.VMEM((IDXG, K), jnp.int32) for _ in range(2)],
            [pltpu.VMEM((K, F), jnp.bfloat16) for _ in range(nbuf)],
            [pltpu.SemaphoreType.DMA for _ in range(nbuf)],
            [pltpu.SemaphoreType.DMA for _ in range(2)],
            pltpu.VMEM_SHARED((NP, F), jnp.bfloat16),
        ],
        compiler_params=pltpu.CompilerParams(use_tc_tiling_on_sc=False),
        name=f"sc_edge_scatter_f{F}",
    )
    def k(g_hbm, src_hbm, dst_hbm, out_hbm, sidx, didx, rows, gsem, isem, acc):
        c = lax.axis_index("c")
        s = lax.axis_index("s")
        w = c * NSUB + s
        cbase = w * NCHUNK

        def idescs(gr, ib):
            off = cbase + gr * IDXG
            return (
                pltpu.make_async_copy(src_hbm.at[pl.ds(off, IDXG), :],
                                      sidx[ib], isem[ib]),
                pltpu.make_async_copy(dst_hbm.at[pl.ds(off, IDXG), :],
                                      didx[ib], isem[ib]),
            )

        for ib in range(2):
            for d_ in idescs(ib, ib):
                d_.start()

        _zero_rows(rows[0], K, F)
        rbase = s * RPT
        for j in range(RPT // K):
            pltpu.sync_copy(rows[0], acc.at[pl.ds(rbase + j * K, K), :])
        plsc.subcore_barrier()

        def gdesc(j, b, ib):
            return pltpu.make_async_copy(g_hbm.at[sidx[ib].at[j]], rows[b],
                                         gsem[b])

        def rungroup(gr, ib):
            for d_ in idescs(gr, ib):
                d_.wait()
            nxt = gr + 2

            @pl.when(nxt < NGRP)
            def _():
                for d_ in idescs(nxt, ib):
                    d_.start()

            for j in range(nbuf):
                gdesc(j, j, ib).start()
            for j in range(IDXG):
                b = j % nbuf
                gdesc(j, b, ib).wait()
                pltpu.sync_copy(rows[b], acc.at[didx[ib].at[j]], add=True)
                if j + nbuf < IDXG:
                    gdesc(j + nbuf, b, ib).start()

        def body(i, _):
            rungroup(2 * i, 0)
            rungroup(2 * i + 1, 1)
            return 0

        lax.fori_loop(0, NGRP // 2, body, 0)
        plsc.subcore_barrier()
        obase = c * NP + rbase
        for j in range(RPT // K):
            pltpu.sync_copy(acc.at[pl.ds(rbase + j * K, K), :], rows[j % 2])
            pltpu.sync_copy(rows[j % 2], out_hbm.at[pl.ds(obase + j * K, K), :])

    return k


DEGQ = 8


@functools.partial(
    pl.kernel,
    mesh=_sc_mesh(),
    out_type=jax.ShapeDtypeStruct((NT,), jnp.float32),
    scratch_types=[
        pltpu.VMEM((NCHUNK, K), jnp.int32),
        pltpu.VMEM((K,), jnp.float32),
        pltpu.VMEM((RPT,), jnp.float32),
        pltpu.SemaphoreType.DMA,
        pltpu.VMEM_SHARED((NP,), jnp.float32),
    ],
    compiler_params=pltpu.CompilerParams(use_tc_tiling_on_sc=False),
    name="sc_degree",
)
def _sc_degree(dst_hbm, out_hbm, dst_all, ones_v, wb, sem, acc):
    c = lax.axis_index("c")
    s = lax.axis_index("s")
    w = c * NSUB + s
    one = jnp.ones((16,), jnp.float32)
    zero = jnp.zeros((16,), jnp.float32)
    pltpu.sync_copy(dst_hbm.at[pl.ds(w * NCHUNK, NCHUNK), :], dst_all)
    for j in range(K // 16):
        ones_v[pl.ds(16 * j, 16)] = one

    def zbody(i, _):
        wb[pl.ds(16 * i, 16)] = zero
        return 0

    lax.fori_loop(0, RPT // 16, zbody, 0)
    rbase = s * RPT
    pltpu.sync_copy(wb, acc.at[pl.ds(rbase, RPT)])
    plsc.subcore_barrier()

    def sdesc(t):
        return pltpu.make_async_copy(ones_v, acc.at[dst_all.at[t]], sem)

    def body(gi, _):
        for b in range(DEGQ):
            pltpu.async_copy(ones_v, acc.at[dst_all.at[gi * DEGQ + b]], sem,
                             add=True)
        for b in range(DEGQ):
            sdesc(gi * DEGQ + b).wait()
        return 0

    lax.fori_loop(0, NCHUNK // DEGQ, body, 0)
    plsc.subcore_barrier()
    pltpu.sync_copy(acc.at[pl.ds(rbase, RPT)], wb)
    pltpu.sync_copy(wb, out_hbm.at[pl.ds(c * NP + rbase, RPT)])


BR = 2048


def _tc_first(x, W, deg):
    Fi, Fo = W.shape

    def body(x_ref, w_ref, deg_ref, g_ref, dis_ref):
        dis = lax.rsqrt(jnp.maximum(deg_ref[...] + 1.0, 1.0))
        g_ref[...] = (dis * jnp.dot(x_ref[...], w_ref[...],
                                    preferred_element_type=jnp.float32)
                      ).astype(jnp.bfloat16)
        dis_ref[...] = dis

    return pl.pallas_call(
        body,
        grid=(NT // BR,),
        in_specs=[
            pl.BlockSpec((BR, Fi), lambda i: (i, 0)),
            pl.BlockSpec((Fi, Fo), lambda i: (0, 0)),
            pl.BlockSpec((BR, 1), lambda i: (i, 0)),
        ],
        out_specs=[
            pl.BlockSpec((BR, Fo), lambda i: (i, 0)),
            pl.BlockSpec((BR, 1), lambda i: (i, 0)),
        ],
        out_shape=[
            jax.ShapeDtypeStruct((NT, Fo), jnp.bfloat16),
            jax.ShapeDtypeStruct((NT, 1), jnp.float32),
        ],
    )(x, W, deg)


def _tc_layer(acc, g, dis, b, W):
    Fi, Fo = W.shape

    def body(acc_ref, g_ref, dis_ref, b_ref, w_ref, out_ref):
        dis = dis_ref[...]
        a = (acc_ref[...].astype(jnp.float32) + g_ref[...].astype(jnp.float32))
        h = jnp.maximum(dis * a + b_ref[...], 0.0)
        out_ref[...] = (dis * jnp.dot(h, w_ref[...],
                                      preferred_element_type=jnp.float32)
                        ).astype(jnp.bfloat16)

    return pl.pallas_call(
        body,
        grid=(NT // BR,),
        in_specs=[
            pl.BlockSpec((BR, Fi), lambda i: (i, 0)),
            pl.BlockSpec((BR, Fi), lambda i: (i, 0)),
            pl.BlockSpec((BR, 1), lambda i: (i, 0)),
            pl.BlockSpec((1, Fi), lambda i: (0, 0)),
            pl.BlockSpec((Fi, Fo), lambda i: (0, 0)),
        ],
        out_specs=pl.BlockSpec((BR, Fo), lambda i: (i, 0)),
        out_shape=jax.ShapeDtypeStruct((NT, Fo), jnp.bfloat16),
    )(acc, g, dis, b, W)


def _tc_head(acc, g, dis, b3, batchp, Wa, Wt2, VtT, bt, Wf, bf, Ws, bs, R, T):
    def body(acc_ref, g_ref, dis_ref, b3_ref, batch_ref, wa_ref, wt2_ref,
             vtt_ref, bt_ref, wf_ref, bf_ref, ws_ref, bs_ref, r_ref, t_ref,
             out_ref):
        a = dis_ref[...] * (acc_ref[...].astype(jnp.float32)
                            + g_ref[...].astype(jnp.float32)) + b3_ref[...]
        seg = lax.broadcasted_iota(jnp.int32, (2 * B, NT), 0)
        oh = (seg == batch_ref[...]).astype(jnp.float32)
        sums = jnp.dot(oh, a, preferred_element_type=jnp.float32)
        cnt = jnp.sum(oh, axis=1, keepdims=True)
        mean = sums / jnp.maximum(cnt, 1.0)
        ctx = jnp.tanh(jnp.dot(mean, wa_ref[...],
                               preferred_element_type=jnp.float32))
        ctx_pn = lax.dot_general(oh, ctx, (((0,), (0,)), ((), ())),
                                 preferred_element_type=jnp.float32)
        score = jax.nn.sigmoid(jnp.sum(a * ctx_pn, axis=1, keepdims=True))
        pooled = jnp.dot(oh, a * score, preferred_element_type=jnp.float32)
        e1 = pooled[:B]
        e2 = pooled[B:]
        op = (jnp.dot(e1, r_ref[...], preferred_element_type=jnp.float32)
              * jnp.dot(e2, t_ref[...], preferred_element_type=jnp.float32))
        scoring = jnp.dot(op, wt2_ref[...], preferred_element_type=jnp.float32)
        blk = jnp.dot(jnp.concatenate([e1, e2], axis=1), vtt_ref[...],
                      preferred_element_type=jnp.float32)
        s = jnp.maximum(scoring + blk + bt_ref[...], 0.0)
        s = jnp.maximum(jnp.dot(s, wf_ref[...],
                                preferred_element_type=jnp.float32)
                        + bf_ref[...], 0.0)
        out_ref[...] = jax.nn.sigmoid(
            jnp.dot(s, ws_ref[...], preferred_element_type=jnp.float32)
            + bs_ref[...])

    return pl.pallas_call(
        body,
        out_shape=jax.ShapeDtypeStruct((B, 1), jnp.float32),
    )(acc, g, dis, b3, batchp, Wa, Wt2, VtT, bt, Wf, bf, Ws, bs, R, T)


def kernel(features_1, edge_index_1, features_2, edge_index_2, batch_1,
           batch_2, W1, b1, W2, b2, W3, b3, Wa, Wt, Vt, bt, Wf, bf, Ws, bs):
    x = jnp.concatenate([
        jnp.pad(features_1, ((0, NP - N), (0, 0))),
        jnp.pad(features_2, ((0, NP - N), (0, 0))),
    ], axis=0)

    padn = EPT - ET
    pad_dst = (jnp.arange(padn, dtype=jnp.int32) % (NP - N)) + N

    def make_edges(ei, gidx):
        srcs = ei[0].reshape(NSUB, ET) + gidx * NP
        dsts = ei[1].reshape(NSUB, ET)
        srcs = jnp.concatenate(
            [srcs, jnp.broadcast_to(pad_dst + gidx * NP, (NSUB, padn))], axis=1)
        dsts = jnp.concatenate(
            [dsts, jnp.broadcast_to(pad_dst, (NSUB, padn))], axis=1)
        return srcs, dsts

    s1, d1 = make_edges(edge_index_1, 0)
    s2, d2 = make_edges(edge_index_2, 1)
    src_idx = jnp.stack([s1, s2]).reshape(2 * NSUB * NCHUNK, K)
    dst_idx = jnp.stack([d1, d2]).reshape(2 * NSUB * NCHUNK, K)

    batchp = jnp.concatenate([
        jnp.pad(batch_1, (0, NP - N), constant_values=-1),
        jnp.pad(batch_2 + B, (0, NP - N), constant_values=-1),
    ]).reshape(1, NT)

    Wt2 = Wt.reshape(32 * 32, 16)
    VtT = Vt.T
    ar = jnp.arange(1024, dtype=jnp.int32)
    a32 = jnp.arange(32, dtype=jnp.int32)
    R = (ar[None, :] // 32 == a32[:, None]).astype(jnp.float32)
    T = (ar[None, :] % 32 == a32[:, None]).astype(jnp.float32)

    deg = _sc_degree(dst_idx).reshape(NT, 1)
    g1, dis = _tc_first(x, W1, deg)
    acc1 = _make_sc_scatter(128)(g1, src_idx, dst_idx)
    g2 = _tc_layer(acc1, g1, dis, b1.reshape(1, -1), W2)
    acc2 = _make_sc_scatter(64)(g2, src_idx, dst_idx)
    g3 = _tc_layer(acc2, g2, dis, b2.reshape(1, -1), W3)
    acc3 = _make_sc_scatter(32)(g3, src_idx, dst_idx)
    score = _tc_head(acc3, g3, dis, b3.reshape(1, -1), batchp, Wa, Wt2, VtT,
                     bt.reshape(1, -1), Wf, bf.reshape(1, -1), Ws,
                     bs.reshape(1, -1), R, T)
    return score.reshape(-1)


# speedup vs baseline: 1.5215x; 1.0776x over previous
import functools

import jax
import jax.numpy as jnp
from jax import lax
from jax.experimental import pallas as pl
from jax.experimental.pallas import tpu as pltpu
from jax.experimental.pallas import tpu_sc as plsc

N = 10000
E = 320000
B = 64
NP = 10240
NT = 2 * NP
NSUB = 16
NCORE = 2
ET = E // NSUB
K = 128
EPT = 20480
NCHUNK = EPT // K
RPT = NP // NSUB


def _sc_mesh():
    return plsc.VectorSubcoreMesh(core_axis_name="c", subcore_axis_name="s")


def _zero_rows(buf, nrows, ncols):
    z = jnp.zeros((32,), jnp.bfloat16)

    def body(i, _):
        for j in range(ncols // 32):
            buf[i, pl.ds(32 * j, 32)] = z
        return 0

    lax.fori_loop(0, nrows, body, 0)


IDXG = 16
NGRP = NCHUNK // IDXG


def _make_sc_scatter(F):
    nbuf = 4

    @functools.partial(
        pl.kernel,
        mesh=_sc_mesh(),
        out_type=jax.ShapeDtypeStruct((NT, F), jnp.bfloat16),
        scratch_types=[
            [pltpu.VMEM((IDXG, K), jnp.int32) for _ in range(2)],
            [pltpu.VMEM((IDXG, K), jnp.int32) for _ in range(2)],
            [pltpu.VMEM((K, F), jnp.bfloat16) for _ in range(nbuf)],
            [pltpu.SemaphoreType.DMA for _ in range(nbuf)],
            [pltpu.SemaphoreType.DMA for _ in range(2)],
            pltpu.SemaphoreType.DMA,
            pltpu.VMEM_SHARED((NP, F), jnp.bfloat16),
        ],
        compiler_params=pltpu.CompilerParams(use_tc_tiling_on_sc=False),
        name=f"sc_edge_scatter_f{F}",
    )
    def k(g_hbm, src_hbm, dst_hbm, out_hbm, sidx, didx, rows, gsem, isem,
          zsem, acc):
        c = lax.axis_index("c")
        s = lax.axis_index("s")
        w = c * NSUB + s
        cbase = w * NCHUNK

        def idescs(gr, ib):
            off = cbase + gr * IDXG
            return (
                pltpu.make_async_copy(src_hbm.at[pl.ds(off, IDXG), :],
                                      sidx[ib], isem[ib]),
                pltpu.make_async_copy(dst_hbm.at[pl.ds(off, IDXG), :],
                                      didx[ib], isem[ib]),
            )

        for ib in range(2):
            for d_ in idescs(ib, ib):
                d_.start()

        _zero_rows(rows[0], K, F)
        rbase = s * RPT
        NZ = RPT // K
        for j in range(NZ):
            pltpu.async_copy(rows[0], acc.at[pl.ds(rbase + j * K, K), :], zsem)
        for j in range(NZ):
            pltpu.make_async_copy(rows[0], acc.at[pl.ds(rbase + j * K, K), :],
                                  zsem).wait()
        plsc.subcore_barrier()

        def gdesc(j, b, ib):
            return pltpu.make_async_copy(g_hbm.at[sidx[ib].at[j]], rows[b],
                                         gsem[b])

        for d_ in idescs(0, 0):
            d_.wait()
        for j in range(nbuf):
            gdesc(j, j, 0).start()

        def rungroup(gr, ib):
            for j in range(IDXG):
                b = j % nbuf
                if j == IDXG - nbuf:
                    @pl.when(gr + 1 < NGRP)
                    def _():
                        for d_ in idescs(gr + 1, 1 - ib):
                            d_.wait()

                gdesc(j, b, ib).wait()
                pltpu.sync_copy(rows[b], acc.at[didx[ib].at[j]], add=True)
                if j + nbuf < IDXG:
                    gdesc(j + nbuf, b, ib).start()
                else:

                    @pl.when(gr + 1 < NGRP)
                    def _():
                        gdesc(j + nbuf - IDXG, b, 1 - ib).start()

            @pl.when(gr + 2 < NGRP)
            def _():
                for d_ in idescs(gr + 2, ib):
                    d_.start()

        def body(i, _):
            rungroup(2 * i, 0)
            rungroup(2 * i + 1, 1)
            return 0

        lax.fori_loop(0, NGRP // 2, body, 0)
        plsc.subcore_barrier()
        obase = c * NP + rbase

        def rdesc(j):
            return pltpu.make_async_copy(acc.at[pl.ds(rbase + j * K, K), :],
                                         rows[j % 2], gsem[j % 2])

        def wdesc(j):
            return pltpu.make_async_copy(rows[j % 2],
                                         out_hbm.at[pl.ds(obase + j * K, K), :],
                                         gsem[2 + j % 2])

        rdesc(0).start()
        for j in range(NZ):
            rdesc(j).wait()
            if j >= 1:
                wdesc(j - 1).wait()
            if j + 1 < NZ:
                rdesc(j + 1).start()
            wdesc(j).start()
        wdesc(NZ - 1).wait()

    return k


DEGQ = 8


@functools.partial(
    pl.kernel,
    mesh=_sc_mesh(),
    out_type=jax.ShapeDtypeStruct((NT,), jnp.float32),
    scratch_types=[
        pltpu.VMEM((NCHUNK, K), jnp.int32),
        pltpu.VMEM((K,), jnp.float32),
        pltpu.VMEM((RPT,), jnp.float32),
        pltpu.SemaphoreType.DMA,
        pltpu.VMEM_SHARED((NP,), jnp.float32),
    ],
    compiler_params=pltpu.CompilerParams(use_tc_tiling_on_sc=False),
    name="sc_degree",
)
def _sc_degree(dst_hbm, out_hbm, dst_all, ones_v, wb, sem, acc):
    c = lax.axis_index("c")
    s = lax.axis_index("s")
    w = c * NSUB + s
    one = jnp.ones((16,), jnp.float32)
    zero = jnp.zeros((16,), jnp.float32)
    pltpu.sync_copy(dst_hbm.at[pl.ds(w * NCHUNK, NCHUNK), :], dst_all)
    for j in range(K // 16):
        ones_v[pl.ds(16 * j, 16)] = one

    def zbody(i, _):
        wb[pl.ds(16 * i, 16)] = zero
        return 0

    lax.fori_loop(0, RPT // 16, zbody, 0)
    rbase = s * RPT
    pltpu.sync_copy(wb, acc.at[pl.ds(rbase, RPT)])
    plsc.subcore_barrier()

    def sdesc(t):
        return pltpu.make_async_copy(ones_v, acc.at[dst_all.at[t]], sem)

    def body(gi, _):
        for b in range(DEGQ):
            pltpu.async_copy(ones_v, acc.at[dst_all.at[gi * DEGQ + b]], sem,
                             add=True)
        for b in range(DEGQ):
            sdesc(gi * DEGQ + b).wait()
        return 0

    lax.fori_loop(0, NCHUNK // DEGQ, body, 0)
    plsc.subcore_barrier()
    pltpu.sync_copy(acc.at[pl.ds(rbase, RPT)], wb)
    pltpu.sync_copy(wb, out_hbm.at[pl.ds(c * NP + rbase, RPT)])


BR = 2048


def _tc_first(x, W, deg):
    Fi, Fo = W.shape

    def body(x_ref, w_ref, deg_ref, g_ref, dis_ref):
        dis = lax.rsqrt(jnp.maximum(deg_ref[...] + 1.0, 1.0))
        g_ref[...] = (dis * jnp.dot(x_ref[...], w_ref[...],
                                    preferred_element_type=jnp.float32)
                      ).astype(jnp.bfloat16)
        dis_ref[...] = dis

    return pl.pallas_call(
        body,
        grid=(NT // BR,),
        in_specs=[
            pl.BlockSpec((BR, Fi), lambda i: (i, 0)),
            pl.BlockSpec((Fi, Fo), lambda i: (0, 0)),
            pl.BlockSpec((BR, 1), lambda i: (i, 0)),
        ],
        out_specs=[
            pl.BlockSpec((BR, Fo), lambda i: (i, 0)),
            pl.BlockSpec((BR, 1), lambda i: (i, 0)),
        ],
        out_shape=[
            jax.ShapeDtypeStruct((NT, Fo), jnp.bfloat16),
            jax.ShapeDtypeStruct((NT, 1), jnp.float32),
        ],
    )(x, W, deg)


def _tc_layer(acc, g, dis, b, W):
    Fi, Fo = W.shape

    def body(acc_ref, g_ref, dis_ref, b_ref, w_ref, out_ref):
        dis = dis_ref[...]
        a = (acc_ref[...].astype(jnp.float32) + g_ref[...].astype(jnp.float32))
        h = jnp.maximum(dis * a + b_ref[...], 0.0)
        out_ref[...] = (dis * jnp.dot(h, w_ref[...],
                                      preferred_element_type=jnp.float32)
                        ).astype(jnp.bfloat16)

    return pl.pallas_call(
        body,
        grid=(NT // BR,),
        in_specs=[
            pl.BlockSpec((BR, Fi), lambda i: (i, 0)),
            pl.BlockSpec((BR, Fi), lambda i: (i, 0)),
            pl.BlockSpec((BR, 1), lambda i: (i, 0)),
            pl.BlockSpec((1, Fi), lambda i: (0, 0)),
            pl.BlockSpec((Fi, Fo), lambda i: (0, 0)),
        ],
        out_specs=pl.BlockSpec((BR, Fo), lambda i: (i, 0)),
        out_shape=jax.ShapeDtypeStruct((NT, Fo), jnp.bfloat16),
    )(acc, g, dis, b, W)


def _tc_head(acc, g, dis, b3, batchp, Wa, Wt2, VtT, bt, Wf, bf, Ws, bs, R, T):
    def body(acc_ref, g_ref, dis_ref, b3_ref, batch_ref, wa_ref, wt2_ref,
             vtt_ref, bt_ref, wf_ref, bf_ref, ws_ref, bs_ref, r_ref, t_ref,
             out_ref):
        a = dis_ref[...] * (acc_ref[...].astype(jnp.float32)
                            + g_ref[...].astype(jnp.float32)) + b3_ref[...]
        seg = lax.broadcasted_iota(jnp.int32, (2 * B, NT), 0)
        oh = (seg == batch_ref[...]).astype(jnp.float32)
        sums = jnp.dot(oh, a, preferred_element_type=jnp.float32)
        cnt = jnp.sum(oh, axis=1, keepdims=True)
        mean = sums / jnp.maximum(cnt, 1.0)
        ctx = jnp.tanh(jnp.dot(mean, wa_ref[...],
                               preferred_element_type=jnp.float32))
        ctx_pn = lax.dot_general(oh, ctx, (((0,), (0,)), ((), ())),
                                 preferred_element_type=jnp.float32)
        score = jax.nn.sigmoid(jnp.sum(a * ctx_pn, axis=1, keepdims=True))
        pooled = jnp.dot(oh, a * score, preferred_element_type=jnp.float32)
        e1 = pooled[:B]
        e2 = pooled[B:]
        op = (jnp.dot(e1, r_ref[...], preferred_element_type=jnp.float32)
              * jnp.dot(e2, t_ref[...], preferred_element_type=jnp.float32))
        scoring = jnp.dot(op, wt2_ref[...], preferred_element_type=jnp.float32)
        blk = jnp.dot(jnp.concatenate([e1, e2], axis=1), vtt_ref[...],
                      preferred_element_type=jnp.float32)
        s = jnp.maximum(scoring + blk + bt_ref[...], 0.0)
        s = jnp.maximum(jnp.dot(s, wf_ref[...],
                                preferred_element_type=jnp.float32)
                        + bf_ref[...], 0.0)
        out_ref[...] = jax.nn.sigmoid(
            jnp.dot(s, ws_ref[...], preferred_element_type=jnp.float32)
            + bs_ref[...])

    return pl.pallas_call(
        body,
        out_shape=jax.ShapeDtypeStruct((B, 1), jnp.float32),
    )(acc, g, dis, b3, batchp, Wa, Wt2, VtT, bt, Wf, bf, Ws, bs, R, T)


def kernel(features_1, edge_index_1, features_2, edge_index_2, batch_1,
           batch_2, W1, b1, W2, b2, W3, b3, Wa, Wt, Vt, bt, Wf, bf, Ws, bs):
    x = jnp.concatenate([
        jnp.pad(features_1, ((0, NP - N), (0, 0))),
        jnp.pad(features_2, ((0, NP - N), (0, 0))),
    ], axis=0)

    padn = EPT - ET
    pad_dst = (jnp.arange(padn, dtype=jnp.int32) % (NP - N)) + N

    def make_edges(ei, gidx):
        srcs = ei[0].reshape(NSUB, ET) + gidx * NP
        dsts = ei[1].reshape(NSUB, ET)
        srcs = jnp.concatenate(
            [srcs, jnp.broadcast_to(pad_dst + gidx * NP, (NSUB, padn))], axis=1)
        dsts = jnp.concatenate(
            [dsts, jnp.broadcast_to(pad_dst, (NSUB, padn))], axis=1)
        return srcs, dsts

    s1, d1 = make_edges(edge_index_1, 0)
    s2, d2 = make_edges(edge_index_2, 1)
    src_idx = jnp.stack([s1, s2]).reshape(2 * NSUB * NCHUNK, K)
    dst_idx = jnp.stack([d1, d2]).reshape(2 * NSUB * NCHUNK, K)

    batchp = jnp.concatenate([
        jnp.pad(batch_1, (0, NP - N), constant_values=-1),
        jnp.pad(batch_2 + B, (0, NP - N), constant_values=-1),
    ]).reshape(1, NT)

    Wt2 = Wt.reshape(32 * 32, 16)
    VtT = Vt.T
    ar = jnp.arange(1024, dtype=jnp.int32)
    a32 = jnp.arange(32, dtype=jnp.int32)
    R = (ar[None, :] // 32 == a32[:, None]).astype(jnp.float32)
    T = (ar[None, :] % 32 == a32[:, None]).astype(jnp.float32)

    deg = _sc_degree(dst_idx).reshape(NT, 1)
    g1, dis = _tc_first(x, W1, deg)
    acc1 = _make_sc_scatter(128)(g1, src_idx, dst_idx)
    g2 = _tc_layer(acc1, g1, dis, b1.reshape(1, -1), W2)
    acc2 = _make_sc_scatter(64)(g2, src_idx, dst_idx)
    g3 = _tc_layer(acc2, g2, dis, b2.reshape(1, -1), W3)
    acc3 = _make_sc_scatter(32)(g3, src_idx, dst_idx)
    score = _tc_head(acc3, g3, dis, b3.reshape(1, -1), batchp, Wa, Wt2, VtT,
                     bt.reshape(1, -1), Wf, bf.reshape(1, -1), Ws,
                     bs.reshape(1, -1), R, T)
    return score.reshape(-1)


# speedup vs baseline: 1.5255x; 1.0027x over previous
import functools

import jax
import jax.numpy as jnp
from jax import lax
from jax.experimental import pallas as pl
from jax.experimental.pallas import tpu as pltpu
from jax.experimental.pallas import tpu_sc as plsc

N = 10000
E = 320000
B = 64
NP = 10240
NT = 2 * NP
NSUB = 16
NCORE = 2
ET = E // NSUB
K = 128
EPT = 20480
NCHUNK = EPT // K
RPT = NP // NSUB


def _sc_mesh():
    return plsc.VectorSubcoreMesh(core_axis_name="c", subcore_axis_name="s")


def _zero_rows(buf, nrows, ncols):
    z = jnp.zeros((32,), jnp.bfloat16)

    def body(i, _):
        for j in range(ncols // 32):
            buf[i, pl.ds(32 * j, 32)] = z
        return 0

    lax.fori_loop(0, nrows, body, 0)


IDXG = 16
NGRP = NCHUNK // IDXG


def _make_sc_scatter(F):
    nbuf = 4

    @functools.partial(
        pl.kernel,
        mesh=_sc_mesh(),
        out_type=jax.ShapeDtypeStruct((NT, F), jnp.bfloat16),
        scratch_types=[
            [pltpu.VMEM((IDXG, K), jnp.int32) for _ in range(2)],
            [pltpu.VMEM((IDXG, K), jnp.int32) for _ in range(2)],
            [pltpu.VMEM((K, F), jnp.bfloat16) for _ in range(nbuf)],
            [pltpu.SemaphoreType.DMA for _ in range(nbuf)],
            [pltpu.SemaphoreType.DMA for _ in range(2)],
            pltpu.SemaphoreType.DMA,
            pltpu.VMEM_SHARED((NP, F), jnp.bfloat16),
        ],
        compiler_params=pltpu.CompilerParams(use_tc_tiling_on_sc=False, skip_device_barrier=True),
        name=f"sc_edge_scatter_f{F}",
    )
    def k(g_hbm, src_hbm, dst_hbm, out_hbm, sidx, didx, rows, gsem, isem,
          zsem, acc):
        c = lax.axis_index("c")
        s = lax.axis_index("s")
        w = c * NSUB + s
        cbase = w * NCHUNK

        def idescs(gr, ib):
            off = cbase + gr * IDXG
            return (
                pltpu.make_async_copy(src_hbm.at[pl.ds(off, IDXG), :],
                                      sidx[ib], isem[ib]),
                pltpu.make_async_copy(dst_hbm.at[pl.ds(off, IDXG), :],
                                      didx[ib], isem[ib]),
            )

        for ib in range(2):
            for d_ in idescs(ib, ib):
                d_.start()

        _zero_rows(rows[0], K, F)
        rbase = s * RPT
        NZ = RPT // K
        for j in range(NZ):
            pltpu.async_copy(rows[0], acc.at[pl.ds(rbase + j * K, K), :], zsem)
        for j in range(NZ):
            pltpu.make_async_copy(rows[0], acc.at[pl.ds(rbase + j * K, K), :],
                                  zsem).wait()
        plsc.subcore_barrier()

        def gdesc(j, b, ib):
            return pltpu.make_async_copy(g_hbm.at[sidx[ib].at[j]], rows[b],
                                         gsem[b])

        for d_ in idescs(0, 0):
            d_.wait()
        for j in range(nbuf):
            gdesc(j, j, 0).start()

        def rungroup(gr, ib):
            for j in range(IDXG):
                b = j % nbuf
                if j == IDXG - nbuf:
                    @pl.when(gr + 1 < NGRP)
                    def _():
                        for d_ in idescs(gr + 1, 1 - ib):
                            d_.wait()

                gdesc(j, b, ib).wait()
                pltpu.sync_copy(rows[b], acc.at[didx[ib].at[j]], add=True)
                if j + nbuf < IDXG:
                    gdesc(j + nbuf, b, ib).start()
                else:

                    @pl.when(gr + 1 < NGRP)
                    def _():
                        gdesc(j + nbuf - IDXG, b, 1 - ib).start()

            @pl.when(gr + 2 < NGRP)
            def _():
                for d_ in idescs(gr + 2, ib):
                    d_.start()

        def body(i, _):
            rungroup(2 * i, 0)
            rungroup(2 * i + 1, 1)
            return 0

        lax.fori_loop(0, NGRP // 2, body, 0)
        plsc.subcore_barrier()
        obase = c * NP + rbase

        def rdesc(j):
            return pltpu.make_async_copy(acc.at[pl.ds(rbase + j * K, K), :],
                                         rows[j % 2], gsem[j % 2])

        def wdesc(j):
            return pltpu.make_async_copy(rows[j % 2],
                                         out_hbm.at[pl.ds(obase + j * K, K), :],
                                         gsem[2 + j % 2])

        rdesc(0).start()
        for j in range(NZ):
            rdesc(j).wait()
            if j >= 1:
                wdesc(j - 1).wait()
            if j + 1 < NZ:
                rdesc(j + 1).start()
            wdesc(j).start()
        wdesc(NZ - 1).wait()

    return k


DEGQ = 8


@functools.partial(
    pl.kernel,
    mesh=_sc_mesh(),
    out_type=jax.ShapeDtypeStruct((NT,), jnp.float32),
    scratch_types=[
        pltpu.VMEM((NCHUNK, K), jnp.int32),
        pltpu.VMEM((K,), jnp.float32),
        pltpu.VMEM((RPT,), jnp.float32),
        pltpu.SemaphoreType.DMA,
        pltpu.VMEM_SHARED((NP,), jnp.float32),
    ],
    compiler_params=pltpu.CompilerParams(use_tc_tiling_on_sc=False, skip_device_barrier=True),
    name="sc_degree",
)
def _sc_degree(dst_hbm, out_hbm, dst_all, ones_v, wb, sem, acc):
    c = lax.axis_index("c")
    s = lax.axis_index("s")
    w = c * NSUB + s
    one = jnp.ones((16,), jnp.float32)
    zero = jnp.zeros((16,), jnp.float32)
    pltpu.sync_copy(dst_hbm.at[pl.ds(w * NCHUNK, NCHUNK), :], dst_all)
    for j in range(K // 16):
        ones_v[pl.ds(16 * j, 16)] = one

    def zbody(i, _):
        wb[pl.ds(16 * i, 16)] = zero
        return 0

    lax.fori_loop(0, RPT // 16, zbody, 0)
    rbase = s * RPT
    pltpu.sync_copy(wb, acc.at[pl.ds(rbase, RPT)])
    plsc.subcore_barrier()

    def sdesc(t):
        return pltpu.make_async_copy(ones_v, acc.at[dst_all.at[t]], sem)

    def body(gi, _):
        for b in range(DEGQ):
            pltpu.async_copy(ones_v, acc.at[dst_all.at[gi * DEGQ + b]], sem,
                             add=True)
        for b in range(DEGQ):
            sdesc(gi * DEGQ + b).wait()
        return 0

    lax.fori_loop(0, NCHUNK // DEGQ, body, 0)
    plsc.subcore_barrier()
    pltpu.sync_copy(acc.at[pl.ds(rbase, RPT)], wb)
    pltpu.sync_copy(wb, out_hbm.at[pl.ds(c * NP + rbase, RPT)])


BR = 2048


def _tc_first(x, W, deg):
    Fi, Fo = W.shape

    def body(x_ref, w_ref, deg_ref, g_ref, dis_ref):
        dis = lax.rsqrt(jnp.maximum(deg_ref[...] + 1.0, 1.0))
        g_ref[...] = (dis * jnp.dot(x_ref[...], w_ref[...],
                                    preferred_element_type=jnp.float32)
                      ).astype(jnp.bfloat16)
        dis_ref[...] = dis

    return pl.pallas_call(
        body,
        grid=(NT // BR,),
        in_specs=[
            pl.BlockSpec((BR, Fi), lambda i: (i, 0)),
            pl.BlockSpec((Fi, Fo), lambda i: (0, 0)),
            pl.BlockSpec((BR, 1), lambda i: (i, 0)),
        ],
        out_specs=[
            pl.BlockSpec((BR, Fo), lambda i: (i, 0)),
            pl.BlockSpec((BR, 1), lambda i: (i, 0)),
        ],
        out_shape=[
            jax.ShapeDtypeStruct((NT, Fo), jnp.bfloat16),
            jax.ShapeDtypeStruct((NT, 1), jnp.float32),
        ],
    )(x, W, deg)


def _tc_layer(acc, g, dis, b, W):
    Fi, Fo = W.shape

    def body(acc_ref, g_ref, dis_ref, b_ref, w_ref, out_ref):
        dis = dis_ref[...]
        a = (acc_ref[...].astype(jnp.float32) + g_ref[...].astype(jnp.float32))
        h = jnp.maximum(dis * a + b_ref[...], 0.0)
        out_ref[...] = (dis * jnp.dot(h, w_ref[...],
                                      preferred_element_type=jnp.float32)
                        ).astype(jnp.bfloat16)

    return pl.pallas_call(
        body,
        grid=(NT // BR,),
        in_specs=[
            pl.BlockSpec((BR, Fi), lambda i: (i, 0)),
            pl.BlockSpec((BR, Fi), lambda i: (i, 0)),
            pl.BlockSpec((BR, 1), lambda i: (i, 0)),
            pl.BlockSpec((1, Fi), lambda i: (0, 0)),
            pl.BlockSpec((Fi, Fo), lambda i: (0, 0)),
        ],
        out_specs=pl.BlockSpec((BR, Fo), lambda i: (i, 0)),
        out_shape=jax.ShapeDtypeStruct((NT, Fo), jnp.bfloat16),
    )(acc, g, dis, b, W)


def _tc_head(acc, g, dis, b3, batchp, Wa, Wt2, VtT, bt, Wf, bf, Ws, bs, R, T):
    def body(acc_ref, g_ref, dis_ref, b3_ref, batch_ref, wa_ref, wt2_ref,
             vtt_ref, bt_ref, wf_ref, bf_ref, ws_ref, bs_ref, r_ref, t_ref,
             out_ref):
        a = dis_ref[...] * (acc_ref[...].astype(jnp.float32)
                            + g_ref[...].astype(jnp.float32)) + b3_ref[...]
        seg = lax.broadcasted_iota(jnp.int32, (2 * B, NT), 0)
        oh = (seg == batch_ref[...]).astype(jnp.float32)
        sums = jnp.dot(oh, a, preferred_element_type=jnp.float32)
        cnt = jnp.sum(oh, axis=1, keepdims=True)
        mean = sums / jnp.maximum(cnt, 1.0)
        ctx = jnp.tanh(jnp.dot(mean, wa_ref[...],
                               preferred_element_type=jnp.float32))
        ctx_pn = lax.dot_general(oh, ctx, (((0,), (0,)), ((), ())),
                                 preferred_element_type=jnp.float32)
        score = jax.nn.sigmoid(jnp.sum(a * ctx_pn, axis=1, keepdims=True))
        pooled = jnp.dot(oh, a * score, preferred_element_type=jnp.float32)
        e1 = pooled[:B]
        e2 = pooled[B:]
        op = (jnp.dot(e1, r_ref[...], preferred_element_type=jnp.float32)
              * jnp.dot(e2, t_ref[...], preferred_element_type=jnp.float32))
        scoring = jnp.dot(op, wt2_ref[...], preferred_element_type=jnp.float32)
        blk = jnp.dot(jnp.concatenate([e1, e2], axis=1), vtt_ref[...],
                      preferred_element_type=jnp.float32)
        s = jnp.maximum(scoring + blk + bt_ref[...], 0.0)
        s = jnp.maximum(jnp.dot(s, wf_ref[...],
                                preferred_element_type=jnp.float32)
                        + bf_ref[...], 0.0)
        out_ref[...] = jax.nn.sigmoid(
            jnp.dot(s, ws_ref[...], preferred_element_type=jnp.float32)
            + bs_ref[...])

    return pl.pallas_call(
        body,
        out_shape=jax.ShapeDtypeStruct((B, 1), jnp.float32),
    )(acc, g, dis, b3, batchp, Wa, Wt2, VtT, bt, Wf, bf, Ws, bs, R, T)


def kernel(features_1, edge_index_1, features_2, edge_index_2, batch_1,
           batch_2, W1, b1, W2, b2, W3, b3, Wa, Wt, Vt, bt, Wf, bf, Ws, bs):
    x = jnp.concatenate([
        jnp.pad(features_1, ((0, NP - N), (0, 0))),
        jnp.pad(features_2, ((0, NP - N), (0, 0))),
    ], axis=0)

    padn = EPT - ET
    pad_dst = (jnp.arange(padn, dtype=jnp.int32) % (NP - N)) + N

    def make_edges(ei, gidx):
        srcs = ei[0].reshape(NSUB, ET) + gidx * NP
        dsts = ei[1].reshape(NSUB, ET)
        srcs = jnp.concatenate(
            [srcs, jnp.broadcast_to(pad_dst + gidx * NP, (NSUB, padn))], axis=1)
        dsts = jnp.concatenate(
            [dsts, jnp.broadcast_to(pad_dst, (NSUB, padn))], axis=1)
        return srcs, dsts

    s1, d1 = make_edges(edge_index_1, 0)
    s2, d2 = make_edges(edge_index_2, 1)
    src_idx = jnp.stack([s1, s2]).reshape(2 * NSUB * NCHUNK, K)
    dst_idx = jnp.stack([d1, d2]).reshape(2 * NSUB * NCHUNK, K)

    batchp = jnp.concatenate([
        jnp.pad(batch_1, (0, NP - N), constant_values=-1),
        jnp.pad(batch_2 + B, (0, NP - N), constant_values=-1),
    ]).reshape(1, NT)

    Wt2 = Wt.reshape(32 * 32, 16)
    VtT = Vt.T
    ar = jnp.arange(1024, dtype=jnp.int32)
    a32 = jnp.arange(32, dtype=jnp.int32)
    R = (ar[None, :] // 32 == a32[:, None]).astype(jnp.float32)
    T = (ar[None, :] % 32 == a32[:, None]).astype(jnp.float32)

    deg = _sc_degree(dst_idx).reshape(NT, 1)
    g1, dis = _tc_first(x, W1, deg)
    acc1 = _make_sc_scatter(128)(g1, src_idx, dst_idx)
    g2 = _tc_layer(acc1, g1, dis, b1.reshape(1, -1), W2)
    acc2 = _make_sc_scatter(64)(g2, src_idx, dst_idx)
    g3 = _tc_layer(acc2, g2, dis, b2.reshape(1, -1), W3)
    acc3 = _make_sc_scatter(32)(g3, src_idx, dst_idx)
    score = _tc_head(acc3, g3, dis, b3.reshape(1, -1), batchp, Wa, Wt2, VtT,
                     bt.reshape(1, -1), Wf, bf.reshape(1, -1), Ws,
                     bs.reshape(1, -1), R, T)
    return score.reshape(-1)
